# baseline (speedup 1.0000x reference)
import functools

import numpy as np
import jax
import jax.numpy as jnp
from jax import lax
from jax.experimental import pallas as pl
from jax.experimental.pallas import tpu as pltpu
from jax.experimental.pallas import tpu_sc as plsc

_MAX_LEN = 2048
_D = 1024

_NC = 2
_NS = 16
_NW = _NC * _NS
_L = 16

_B_TOTAL = 4 * _MAX_LEN
_PER_W = _B_TOTAL // _NW
_CH = 32
_NCH = _PER_W // _CH


def _positional_encoding() -> np.ndarray:
    pos = np.arange(_MAX_LEN, dtype=np.float32)[:, None]
    i = np.arange(0, _D, 2, dtype=np.float32)
    div = np.exp(-np.log(10000.0) * i / _D)
    pe = np.zeros((_MAX_LEN, _D), dtype=np.float32)
    pe[:, 0::2] = np.sin(pos * div)
    pe[:, 1::2] = np.cos(pos * div)
    return pe


_POS = _positional_encoding()

_mesh = plsc.VectorSubcoreMesh(core_axis_name="c", subcore_axis_name="s")


@functools.partial(
    pl.kernel,
    mesh=_mesh,
    out_type=jax.ShapeDtypeStruct((_B_TOTAL, _D), jnp.float32),
    scratch_types=[
        pltpu.VMEM((_PER_W,), jnp.int32),
        pltpu.VMEM((_CH, _D), jnp.float32),
        pltpu.VMEM((_CH, _D), jnp.float32),
        pltpu.SemaphoreType.DMA,
        pltpu.SemaphoreType.DMA,
    ],
)
def _emb_kernel(x_hbm, pos_hbm, table_hbm, out_hbm, idx_v, rows_v, pos_v,
                gsem, psem):
    wid = lax.axis_index("s") * _NC + lax.axis_index("c")
    base = wid * _PER_W
    pbase = base % _MAX_LEN

    pltpu.sync_copy(x_hbm.at[pl.ds(base, _PER_W)], idx_v)

    def chunk(c, carry):
        off = c * _CH
        g = pltpu.async_copy(table_hbm.at[idx_v.at[pl.ds(off, _CH)]],
                             rows_v, gsem)
        p = pltpu.async_copy(pos_hbm.at[pl.ds(pbase + off, _CH)], pos_v, psem)
        g.wait()
        p.wait()

        def row(i, carry2):
            for j in range(_D // _L):
                sl = pl.ds(j * _L, _L)
                rows_v[i, sl] = rows_v[i, sl] + pos_v[i, sl]
            return carry2

        lax.fori_loop(0, _CH, row, 0)
        pltpu.sync_copy(rows_v, out_hbm.at[pl.ds(base + off, _CH)])
        return carry

    lax.fori_loop(0, _NCH, chunk, 0)


def kernel(x, start_token, end_token, table):
    batch, seq_len = x.shape
    out = _emb_kernel(x.reshape(-1), jnp.asarray(_POS), table)
    return out.reshape(batch, seq_len, _D)


# speedup vs baseline: 1.3925x; 1.3925x over previous
import functools

import numpy as np
import jax
import jax.numpy as jnp
from jax import lax
from jax.experimental import pallas as pl
from jax.experimental.pallas import tpu as pltpu
from jax.experimental.pallas import tpu_sc as plsc

_MAX_LEN = 2048
_D = 1024

_NC = 2
_NS = 16
_NW = _NC * _NS
_L = 16

_B_TOTAL = 4 * _MAX_LEN
_PER_W = _B_TOTAL // _NW
_CH = 16
_NCH = _PER_W // _CH
_NG = 2
_NP = 3


def _positional_encoding() -> np.ndarray:
    pos = np.arange(_MAX_LEN, dtype=np.float32)[:, None]
    i = np.arange(0, _D, 2, dtype=np.float32)
    div = np.exp(-np.log(10000.0) * i / _D)
    pe = np.zeros((_MAX_LEN, _D), dtype=np.float32)
    pe[:, 0::2] = np.sin(pos * div)
    pe[:, 1::2] = np.cos(pos * div)
    return pe


_POS = _positional_encoding()

_mesh = plsc.VectorSubcoreMesh(core_axis_name="c", subcore_axis_name="s")


@functools.partial(
    pl.kernel,
    mesh=_mesh,
    out_type=jax.ShapeDtypeStruct((_B_TOTAL, _D), jnp.float32),
    scratch_types=(
        [pltpu.VMEM((_PER_W,), jnp.int32)]
        + [pltpu.VMEM((_CH, _D), jnp.float32)] * (_NG + _NP)
        + [pltpu.SemaphoreType.DMA] * (_NG + 2 * _NP)
    ),
)
def _emb_kernel(x_hbm, pos_hbm, table_hbm, out_hbm, idx_v,
                rows0, rows1, pbuf0, pbuf1, pbuf2,
                gsem0, gsem1, psem0, psem1, psem2, osem0, osem1, osem2):
    rows = (rows0, rows1)
    pbuf = (pbuf0, pbuf1, pbuf2)
    gsem = (gsem0, gsem1)
    psem = (psem0, psem1, psem2)
    osem = (osem0, osem1, osem2)

    wid = lax.axis_index("s") * _NC + lax.axis_index("c")
    base = wid * _PER_W
    pbase = base % _MAX_LEN

    pltpu.sync_copy(x_hbm.at[pl.ds(base, _PER_W)], idx_v)

    def fire_gather(c):
        return pltpu.async_copy(
            table_hbm.at[idx_v.at[pl.ds(c * _CH, _CH)]],
            rows[c % _NG], gsem[c % _NG])

    def fire_pos(c):
        return pltpu.async_copy(
            pos_hbm.at[pl.ds(pbase + c * _CH, _CH)],
            pbuf[c % _NP], psem[c % _NP])

    g_cp = [None] * _NCH
    p_cp = [None] * _NCH
    o_cp = [None] * _NCH
    for c in range(_NG):
        g_cp[c] = fire_gather(c)
    for c in range(_NP):
        p_cp[c] = fire_pos(c)

    for c in range(_NCH):
        g_cp[c].wait()
        p_cp[c].wait()

        pb = pbuf[c % _NP]
        rb = rows[c % _NG]

        def row(i, carry, pb=pb, rb=rb):
            for j in range(_D // _L):
                sl = pl.ds(j * _L, _L)
                plsc.addupdate(pb.at[i, sl], rb[i, sl])
            return carry

        lax.fori_loop(0, _CH, row, 0)

        o_cp[c] = pltpu.async_copy(pb, out_hbm.at[pl.ds(base + c * _CH, _CH)],
                                   osem[c % _NP])
        if c + _NG < _NCH:
            g_cp[c + _NG] = fire_gather(c + _NG)
        if c >= 1 and c + _NG < _NCH:
            o_cp[c - 1].wait()
            p_cp[c + _NG] = fire_pos(c + _NG)

    for c in range(_NCH - _NP, _NCH):
        o_cp[c].wait()


def kernel(x, start_token, end_token, table):
    batch, seq_len = x.shape
    out = _emb_kernel(x.reshape(-1), jnp.asarray(_POS), table)
    return out.reshape(batch, seq_len, _D)
